# Initial kernel scaffold; baseline (speedup 1.0000x reference)
import jax
import jax.numpy as jnp
from jax.experimental import pallas as pl


def kernel(table, pos_ix0, pos_ix1, neg_ix0, neg_ix1):
    raise NotImplementedError("write your pallas kernel here")


# speedup vs baseline: 5.4340x; 5.4340x over previous
import functools

import jax
import jax.numpy as jnp
from jax import lax
from jax.experimental import pallas as pl
from jax.experimental.pallas import tpu as pltpu
from jax.experimental.pallas import tpu_sc as plsc

_POS = 65536
_NEG = 262144
_DIM = 64
_NC = 2
_NS = 16
_NW = _NC * _NS
_LANES = 16
_CH = 128
_POS_W = _POS // _NW
_NEG_W = _NEG // _NW
_POS_MARGIN = 0.1
_NEG_MARGIN = 1.0


def _sqrt16(x):
    bits = lax.bitcast_convert_type(x, jnp.int32)
    r = lax.bitcast_convert_type(jnp.int32(0x5F3759DF) - (bits >> 1),
                                 jnp.float32)
    for _ in range(3):
        r = r * (1.5 - 0.5 * x * r * r)
    return x * r


def _make_sc_kernel():
    mesh = plsc.VectorSubcoreMesh(
        core_axis_name="c", subcore_axis_name="s", num_cores=_NC,
        num_subcores=_NS)

    @functools.partial(
        pl.kernel,
        out_type=jax.ShapeDtypeStruct((_NW, _LANES), jnp.float32),
        mesh=mesh,
        compiler_params=pltpu.CompilerParams(use_tc_tiling_on_sc=False),
        scratch_types=[
            pltpu.VMEM((_NEG_W,), jnp.int32),
            pltpu.VMEM((_NEG_W,), jnp.int32),
            pltpu.VMEM((_CH, _DIM), jnp.float32),
            pltpu.VMEM((_CH, _DIM), jnp.float32),
            pltpu.VMEM((_LANES,), jnp.float32),
            pltpu.SemaphoreType.DMA,
            pltpu.SemaphoreType.DMA,
        ],
    )
    def sc_kernel(table_hbm, pix0_hbm, pix1_hbm, nix0_hbm, nix1_hbm,
                  out_hbm, idx0_v, idx1_v, rows_a, rows_b, acc_v,
                  sem_a, sem_b):
        wid = lax.axis_index("s") * _NC + lax.axis_index("c")
        lane = lax.iota(jnp.int32, _LANES)

        def phase(ix0_hbm, ix1_hbm, n_per_w, is_pos, acc):
            base = wid * n_per_w
            pltpu.sync_copy(ix0_hbm.at[pl.ds(base, n_per_w)],
                            idx0_v.at[pl.ds(0, n_per_w)])
            pltpu.sync_copy(ix1_hbm.at[pl.ds(base, n_per_w)],
                            idx1_v.at[pl.ds(0, n_per_w)])

            def chunk_body(c, acc):
                cp_a = pltpu.async_copy(
                    table_hbm.at[idx0_v.at[pl.ds(c * _CH, _CH)]], rows_a,
                    sem_a)
                cp_b = pltpu.async_copy(
                    table_hbm.at[idx1_v.at[pl.ds(c * _CH, _CH)]], rows_b,
                    sem_b)
                cp_a.wait()
                cp_b.wait()

                def group_body(g, acc):
                    gbase = g * _LANES
                    sv = jnp.zeros((_LANES,), jnp.float32)
                    for j in range(_LANES):
                        p = gbase + j
                        s = None
                        for k in range(_DIM // _LANES):
                            va = rows_a[p, pl.ds(k * _LANES, _LANES)]
                            vb = rows_b[p, pl.ds(k * _LANES, _LANES)]
                            df = va - vb
                            s = df * df if s is None else s + df * df
                        vals = [s[l] for l in range(_LANES)]
                        while len(vals) > 1:
                            vals = [vals[i] + vals[i + 1]
                                    for i in range(0, len(vals), 2)]
                        sv = jnp.where(lane == j, vals[0], sv)
                    dist = _sqrt16(sv + 1e-12)
                    if is_pos:
                        t = jnp.maximum(dist - _POS_MARGIN, 0.0)
                    else:
                        t = jnp.maximum(_NEG_MARGIN - dist, 0.0)
                    return acc + t * t

                return lax.fori_loop(0, _CH // _LANES, group_body, acc)

            return lax.fori_loop(0, n_per_w // _CH, chunk_body, acc)

        acc = jnp.zeros((_LANES,), jnp.float32)
        acc = phase(pix0_hbm, pix1_hbm, _POS_W, True, acc)
        acc = phase(nix0_hbm, nix1_hbm, _NEG_W, False, acc)
        acc_v[...] = acc
        pltpu.sync_copy(acc_v, out_hbm.at[wid])

    return sc_kernel


_SC_KERNEL = _make_sc_kernel()


def kernel(table, pos_ix0, pos_ix1, neg_ix0, neg_ix1):
    parts = _SC_KERNEL(table,
                       pos_ix0.astype(jnp.int32), pos_ix1.astype(jnp.int32),
                       neg_ix0.astype(jnp.int32), neg_ix1.astype(jnp.int32))
    return jnp.sum(parts)


# speedup vs baseline: 8.8855x; 1.6352x over previous
import functools

import jax
import jax.numpy as jnp
from jax import lax
from jax.experimental import pallas as pl
from jax.experimental.pallas import tpu as pltpu
from jax.experimental.pallas import tpu_sc as plsc

_POS = 65536
_NEG = 262144
_DIM = 64
_NC = 2
_NS = 16
_NW = _NC * _NS
_LANES = 16
_CH = 128
_POS_W = _POS // _NW
_NEG_W = _NEG // _NW
_POS_MARGIN = 0.1
_NEG_MARGIN = 1.0


def _sqrt16(x):
    bits = lax.bitcast_convert_type(x, jnp.int32)
    r = lax.bitcast_convert_type(jnp.int32(0x5F3759DF) - (bits >> 1),
                                 jnp.float32)
    for _ in range(3):
        r = r * (1.5 - 0.5 * x * r * r)
    return x * r


def _make_sc_kernel():
    mesh = plsc.VectorSubcoreMesh(
        core_axis_name="c", subcore_axis_name="s", num_cores=_NC,
        num_subcores=_NS)

    @functools.partial(
        pl.kernel,
        out_type=jax.ShapeDtypeStruct((_NW, _LANES), jnp.float32),
        mesh=mesh,
        compiler_params=pltpu.CompilerParams(use_tc_tiling_on_sc=False),
        scratch_types=[
            pltpu.VMEM((_NEG_W,), jnp.int32),
            pltpu.VMEM((_NEG_W,), jnp.int32),
            pltpu.VMEM((_CH, _DIM), jnp.float32),
            pltpu.VMEM((_CH, _DIM), jnp.float32),
            pltpu.VMEM((_CH, _DIM), jnp.float32),
            pltpu.VMEM((_CH, _DIM), jnp.float32),
            pltpu.VMEM((_LANES,), jnp.float32),
            pltpu.SemaphoreType.DMA,
            pltpu.SemaphoreType.DMA,
            pltpu.SemaphoreType.DMA,
            pltpu.SemaphoreType.DMA,
        ],
    )
    def sc_kernel(table_hbm, pix0_hbm, pix1_hbm, nix0_hbm, nix1_hbm,
                  out_hbm, idx0_v, idx1_v, rows_a0, rows_b0, rows_a1,
                  rows_b1, acc_v, sem_a0, sem_b0, sem_a1, sem_b1):
        wid = lax.axis_index("s") * _NC + lax.axis_index("c")
        lane = lax.iota(jnp.int32, _LANES)
        bufs = ((rows_a0, rows_b0, sem_a0, sem_b0),
                (rows_a1, rows_b1, sem_a1, sem_b1))

        def phase(ix0_hbm, ix1_hbm, n_per_w, is_pos, acc):
            base = wid * n_per_w
            pltpu.sync_copy(ix0_hbm.at[pl.ds(base, n_per_w)],
                            idx0_v.at[pl.ds(0, n_per_w)])
            pltpu.sync_copy(ix1_hbm.at[pl.ds(base, n_per_w)],
                            idx1_v.at[pl.ds(0, n_per_w)])
            nch2 = n_per_w // _CH // 2

            def start(c, bi):
                ra, rb, sa, sb = bufs[bi]
                pltpu.async_copy(
                    table_hbm.at[idx0_v.at[pl.ds(c * _CH, _CH)]], ra, sa)
                pltpu.async_copy(
                    table_hbm.at[idx1_v.at[pl.ds(c * _CH, _CH)]], rb, sb)

            def wait(bi):
                ra, rb, sa, sb = bufs[bi]
                pltpu.make_async_copy(
                    table_hbm.at[idx0_v.at[pl.ds(0, _CH)]], ra, sa).wait()
                pltpu.make_async_copy(
                    table_hbm.at[idx1_v.at[pl.ds(0, _CH)]], rb, sb).wait()

            def compute(bi, is_pos, acc):
                ra, rb, _, _ = bufs[bi]

                def group_body(g, acc):
                    gbase = g * _LANES
                    sv = jnp.zeros((_LANES,), jnp.float32)
                    for j in range(_LANES):
                        p = gbase + j
                        s = None
                        for k in range(_DIM // _LANES):
                            va = ra[p, pl.ds(k * _LANES, _LANES)]
                            vb = rb[p, pl.ds(k * _LANES, _LANES)]
                            df = va - vb
                            s = df * df if s is None else s + df * df
                        s = s + lax.rev(s, (0,))
                        vals = [s[l] for l in range(_LANES // 2)]
                        while len(vals) > 1:
                            vals = [vals[i] + vals[i + 1]
                                    for i in range(0, len(vals), 2)]
                        sv = jnp.where(lane == j, vals[0], sv)
                    dist = _sqrt16(sv + 1e-12)
                    if is_pos:
                        t = jnp.maximum(dist - _POS_MARGIN, 0.0)
                    else:
                        t = jnp.maximum(_NEG_MARGIN - dist, 0.0)
                    return acc + t * t

                return lax.fori_loop(0, _CH // _LANES, group_body, acc)

            start(0, 0)

            def chunk2_body(cc, acc):
                c0 = 2 * cc
                start(c0 + 1, 1)
                wait(0)
                acc = compute(0, is_pos, acc)

                @pl.when(cc + 1 < nch2)
                def _():
                    start(c0 + 2, 0)

                wait(1)
                acc = compute(1, is_pos, acc)
                return acc

            return lax.fori_loop(0, nch2, chunk2_body, acc)

        acc = jnp.zeros((_LANES,), jnp.float32)
        acc = phase(pix0_hbm, pix1_hbm, _POS_W, True, acc)
        acc = phase(nix0_hbm, nix1_hbm, _NEG_W, False, acc)
        acc_v[...] = acc
        pltpu.sync_copy(acc_v, out_hbm.at[wid])

    return sc_kernel


_SC_KERNEL = _make_sc_kernel()


def kernel(table, pos_ix0, pos_ix1, neg_ix0, neg_ix1):
    parts = _SC_KERNEL(table,
                       pos_ix0.astype(jnp.int32), pos_ix1.astype(jnp.int32),
                       neg_ix0.astype(jnp.int32), neg_ix1.astype(jnp.int32))
    return jnp.sum(parts)
